# Initial kernel scaffold; baseline (speedup 1.0000x reference)
import jax
import jax.numpy as jnp
from jax.experimental import pallas as pl


def kernel(x, species_table, ability_table, item_table, move_table, group_idx):
    raise NotImplementedError("write your pallas kernel here")


# speedup vs baseline: 3.2727x; 3.2727x over previous
import functools

import jax
import jax.numpy as jnp
from jax import lax
from jax.experimental import pallas as pl
from jax.experimental.pallas import tpu as pltpu
from jax.experimental.pallas import tpu_sc as plsc

NC, NS = 2, 16
NW = NC * NS
CHUNK = 128

_PIECES = [(0, 32, 0), (32, 16, 1), (48, 16, 2),
           (64, 16, 3), (80, 16, 3), (96, 16, 3), (112, 16, 3)]
_DOUT = 153


def _body(xp_hbm, idx_hbm, sp_hbm, ab_hbm, it_hbm, mv_hbm, out_hbm,
          idx_v, sp_buf, sm_buf, gsem, ssem, psem, *, n_rows):
    bpw = n_rows // NW
    g_steps = bpw // CHUNK
    wid = lax.axis_index("s") * NC + lax.axis_index("c")
    base_w = wid * bpw

    tables = [sp_hbm, ab_hbm, it_hbm, mv_hbm]

    pass_cp = pltpu.make_async_copy(
        xp_hbm.at[pl.ds(base_w, bpw)],
        out_hbm.at[pl.ds(base_w, bpw), pl.ds(128, 25)],
        psem)
    pass_cp.start()

    pltpu.sync_copy(idx_hbm.at[wid], idx_v)

    def step(g, carry):
        base = base_w + g * CHUNK
        gcps = []
        for k, (_, _, t) in enumerate(_PIECES):
            dst = sp_buf if k == 0 else sm_buf.at[k - 1]
            gcps.append(pltpu.make_async_copy(
                tables[t].at[idx_v.at[k, g]], dst, gsem))
        for cp in gcps:
            cp.start()
        for cp in gcps:
            cp.wait()
        scps = []
        for k, (col, w, _) in enumerate(_PIECES):
            src = sp_buf if k == 0 else sm_buf.at[k - 1]
            scps.append(pltpu.make_async_copy(
                src, out_hbm.at[pl.ds(base, CHUNK), pl.ds(col, w)], ssem))
        for cp in scps:
            cp.start()
        for cp in scps:
            cp.wait()
        return carry

    lax.fori_loop(0, g_steps, step, 0)
    pass_cp.wait()


def kernel(x, species_table, ability_table, item_table, move_table,
           group_idx=0):
    b, s, f = x.shape
    n = b * s
    x2 = x.reshape(n, f)

    ids = jnp.clip(x2[:, :7].astype(jnp.int32), 0, None)
    caps = jnp.array(
        [species_table.shape[0] - 1, ability_table.shape[0] - 1,
         item_table.shape[0] - 1] + [move_table.shape[0] - 1] * 4,
        jnp.int32)
    ids = jnp.minimum(ids, caps[None, :])
    bpw = n // NW
    g_steps = bpw // CHUNK
    idx = ids.T.reshape(7, NW, g_steps, CHUNK).transpose(1, 0, 2, 3)

    run = functools.partial(
        pl.kernel,
        out_type=jax.ShapeDtypeStruct((n, _DOUT), jnp.float32),
        mesh=plsc.VectorSubcoreMesh(core_axis_name="c", subcore_axis_name="s"),
        scratch_types=[
            pltpu.VMEM((7, g_steps, CHUNK), jnp.int32),
            pltpu.VMEM((CHUNK, 32), jnp.float32),
            pltpu.VMEM((6, CHUNK, 16), jnp.float32),
            pltpu.SemaphoreType.DMA,
            pltpu.SemaphoreType.DMA,
            pltpu.SemaphoreType.DMA,
        ],
        compiler_params=pltpu.CompilerParams(use_tc_tiling_on_sc=False),
    )(functools.partial(_body, n_rows=n))

    xpass = x2[:, 7:32]
    out = run(xpass, idx, species_table, ability_table, item_table,
              move_table)
    return out.reshape(b, s, _DOUT)


# speedup vs baseline: 3.2736x; 1.0003x over previous
import functools

import jax
import jax.numpy as jnp
from jax import lax
from jax.experimental import pallas as pl
from jax.experimental.pallas import tpu as pltpu
from jax.experimental.pallas import tpu_sc as plsc

NC, NS = 2, 16
NW = NC * NS
CHUNK = 128
NBUF = 5

_PIECES = [(0, 32, 0), (32, 16, 1), (48, 16, 2),
           (64, 16, 3), (80, 16, 3), (96, 16, 3), (112, 16, 3)]
_DOUT = 153


def _body(xp_hbm, idx_hbm, sp_hbm, ab_hbm, it_hbm, mv_hbm, out_hbm,
          idx_v, sp_bufs, sm_bufs, psem, *sems, n_rows):
    bpw = n_rows // NW
    g_steps = bpw // CHUNK
    wid = lax.axis_index("s") * NC + lax.axis_index("c")
    base_w = wid * bpw

    tables = [sp_hbm, ab_hbm, it_hbm, mv_hbm]
    gsems = sems[:NBUF]
    ssems = sems[NBUF:]

    pass_cp = pltpu.make_async_copy(
        xp_hbm.at[pl.ds(base_w, bpw)],
        out_hbm.at[pl.ds(base_w, bpw), pl.ds(128, 25)],
        psem)
    pass_cp.start()

    pltpu.sync_copy(idx_hbm.at[wid], idx_v)

    def gather_cps(b, g):
        cps = []
        for k, (_, _, t) in enumerate(_PIECES):
            dst = sp_bufs.at[b] if k == 0 else sm_bufs.at[b, k - 1]
            cps.append(pltpu.make_async_copy(
                tables[t].at[idx_v.at[k, g]], dst, gsems[b]))
        return cps

    def store_cps(b, g):
        base = base_w + g * CHUNK
        cps = []
        for k, (col, w, _) in enumerate(_PIECES):
            src = sp_bufs.at[b] if k == 0 else sm_bufs.at[b, k - 1]
            cps.append(pltpu.make_async_copy(
                src, out_hbm.at[pl.ds(base, CHUNK), pl.ds(col, w)],
                ssems[b]))
        return cps

    for b in range(NBUF):
        for cp in gather_cps(b, b):
            cp.start()

    def outer(o, carry):
        for b in range(NBUF):
            g = o * NBUF + b
            for cp in gather_cps(b, 0):
                cp.wait()
            for cp in store_cps(b, g):
                cp.start()
            for cp in store_cps(b, 0):
                cp.wait()
            nxt = g + NBUF

            @pl.when(nxt < g_steps)
            def _():
                for cp in gather_cps(b, nxt):
                    cp.start()
        return carry

    lax.fori_loop(0, g_steps // NBUF, outer, 0)
    pass_cp.wait()


def kernel(x, species_table, ability_table, item_table, move_table,
           group_idx=0):
    b, s, f = x.shape
    n = b * s
    x2 = x.reshape(n, f)

    ids = jnp.clip(x2[:, :7].astype(jnp.int32), 0, None)
    caps = jnp.array(
        [species_table.shape[0] - 1, ability_table.shape[0] - 1,
         item_table.shape[0] - 1] + [move_table.shape[0] - 1] * 4,
        jnp.int32)
    ids = jnp.minimum(ids, caps[None, :])
    bpw = n // NW
    g_steps = bpw // CHUNK
    idx = ids.T.reshape(7, NW, g_steps, CHUNK).transpose(1, 0, 2, 3)

    run = functools.partial(
        pl.kernel,
        out_type=jax.ShapeDtypeStruct((n, _DOUT), jnp.float32),
        mesh=plsc.VectorSubcoreMesh(core_axis_name="c", subcore_axis_name="s"),
        scratch_types=[
            pltpu.VMEM((7, g_steps, CHUNK), jnp.int32),
            pltpu.VMEM((NBUF, CHUNK, 32), jnp.float32),
            pltpu.VMEM((NBUF, 6, CHUNK, 16), jnp.float32),
            pltpu.SemaphoreType.DMA,
        ] + [pltpu.SemaphoreType.DMA] * (2 * NBUF),
        compiler_params=pltpu.CompilerParams(use_tc_tiling_on_sc=False),
    )(functools.partial(_body, n_rows=n))

    xpass = x2[:, 7:32]
    out = run(xpass, idx, species_table, ability_table, item_table,
              move_table)
    return out.reshape(b, s, _DOUT)
